# Initial kernel scaffold; baseline (speedup 1.0000x reference)
import functools
import math

import jax
import jax.numpy as jnp
from jax import lax
from jax.experimental import pallas as pl
from jax.experimental.pallas import tpu as pltpu
from jax.experimental.pallas import tpu_sc as plsc

EPS = 1e-8
NCOMP = 16
LANES = 16
NC, NS = 2, 16
NW = NC * NS
N = 1024 * 2048
PER_W = N // NW
CHUNK = 8192
NCHUNK = PER_W // CHUNK
VPC = CHUNK // LANES


def _sc_gmm(w_flat, mu_b, nis_b, coef_b, invt_b):
    mesh = plsc.VectorSubcoreMesh(core_axis_name="c", subcore_axis_name="s")

    @functools.partial(
        pl.kernel,
        mesh=mesh,
        out_type=jax.ShapeDtypeStruct((N,), jnp.float32),
        scratch_types=[
            pltpu.VMEM((NCOMP, LANES), jnp.float32),
            pltpu.VMEM((NCOMP, LANES), jnp.float32),
            pltpu.VMEM((NCOMP, LANES), jnp.float32),
            pltpu.VMEM((LANES,), jnp.float32),
            pltpu.VMEM((CHUNK,), jnp.float32),
            pltpu.VMEM((CHUNK,), jnp.float32),
        ],
    )
    def body(w_hbm, mu_hbm, nis_hbm, coef_hbm, invt_hbm, out_hbm,
             mu_v, nis_v, coef_v, invt_v, wbuf, obuf):
        wid = lax.axis_index("s") * NC + lax.axis_index("c")
        base = wid * PER_W
        pltpu.sync_copy(mu_hbm, mu_v)
        pltpu.sync_copy(nis_hbm, nis_v)
        pltpu.sync_copy(coef_hbm, coef_v)
        pltpu.sync_copy(invt_hbm, invt_v)

        invt = invt_v[...]

        def chunk_body(j, carry):
            off = base + j * CHUNK
            pltpu.sync_copy(w_hbm.at[pl.ds(off, CHUNK)], wbuf)

            def vec_body(i, c2):
                w = wbuf[pl.ds(i * LANES, LANES)]
                us = []
                s = None
                m = None
                for k in range(NCOMP):
                    d = w - mu_v[k]
                    u = coef_v[k] * jnp.exp(d * d * nis_v[k])
                    us.append(u)
                    if k == 0:
                        s = u
                        m = u
                    else:
                        s = s + u
                        m = jnp.maximum(m, u)
                c = invt / (s + EPS)
                den = None
                num = None
                for k in range(NCOMP):
                    e = jnp.exp((us[k] - m) * c)
                    if k == 0:
                        den = e
                    else:
                        den = den + e
                        num = e * mu_v[k] if k == 1 else num + e * mu_v[k]
                obuf[pl.ds(i * LANES, LANES)] = num / den
                return c2

            lax.fori_loop(0, VPC, vec_body, 0, unroll=2)
            pltpu.sync_copy(obuf, out_hbm.at[pl.ds(off, CHUNK)])
            return carry

        lax.fori_loop(0, NCHUNK, chunk_body, 0)

    return body(w_flat, mu_b, nis_b, coef_b, invt_b)


def kernel(weights, mu, pi_k, pi_zero, sigma, sigma_zero, temperature):
    w = weights.reshape(-1)
    pi_tmp = jnp.abs(jnp.concatenate([pi_zero, pi_k]))
    pi_norm = pi_tmp / jnp.sum(pi_tmp)
    mu_all = jnp.concatenate([jnp.zeros((1,), weights.dtype), mu])
    sigma_all = jnp.concatenate([sigma_zero, sigma])
    two_sig2 = 2.0 * sigma_all ** 2
    coef = pi_norm / jnp.sqrt(math.pi * two_sig2)
    nis = -1.0 / two_sig2

    mu_b = jnp.broadcast_to(mu_all[:, None], (NCOMP, LANES))
    nis_b = jnp.broadcast_to(nis[:, None], (NCOMP, LANES))
    coef_b = jnp.broadcast_to(coef[:, None], (NCOMP, LANES))
    invt_b = jnp.broadcast_to(1.0 / temperature, (LANES,))

    out = _sc_gmm(w, mu_b, nis_b, coef_b, invt_b)
    return out.reshape(weights.shape)


# speedup vs baseline: 1.6535x; 1.6535x over previous
import functools
import math

import jax
import jax.numpy as jnp
from jax import lax
from jax.experimental import pallas as pl
from jax.experimental.pallas import tpu as pltpu
from jax.experimental.pallas import tpu_sc as plsc

EPS = 1e-8
NCOMP = 16
LANES = 16
NC, NS = 2, 16
NW = NC * NS
N = 1024 * 2048
PER_W = N // NW
CHUNK = 8192
NCHUNK = PER_W // CHUNK
VPC = CHUNK // LANES


def _sc_gmm(w_flat, mu_b, nis_b, coef_b, invt_b):
    mesh = plsc.VectorSubcoreMesh(core_axis_name="c", subcore_axis_name="s")

    @functools.partial(
        pl.kernel,
        mesh=mesh,
        out_type=jax.ShapeDtypeStruct((N,), jnp.float32),
        scratch_types=[
            pltpu.VMEM((NCOMP, LANES), jnp.float32),
            pltpu.VMEM((NCOMP, LANES), jnp.float32),
            pltpu.VMEM((NCOMP, LANES), jnp.float32),
            pltpu.VMEM((LANES,), jnp.float32),
            pltpu.VMEM((CHUNK,), jnp.float32),
            pltpu.VMEM((CHUNK,), jnp.float32),
        ],
    )
    def body(w_hbm, mu_hbm, nis_hbm, coef_hbm, invt_hbm, out_hbm,
             mu_v, nis_v, coef_v, invt_v, wbuf, obuf):
        wid = lax.axis_index("s") * NC + lax.axis_index("c")
        base = wid * PER_W
        pltpu.sync_copy(mu_hbm, mu_v)
        pltpu.sync_copy(nis_hbm, nis_v)
        pltpu.sync_copy(coef_hbm, coef_v)
        pltpu.sync_copy(invt_hbm, invt_v)

        @pl.loop(0, NCHUNK)
        def chunk_body(j):
            off = base + j * CHUNK
            pltpu.sync_copy(w_hbm.at[pl.ds(off, CHUNK)], wbuf)

            @pl.loop(0, VPC)
            def vec_body(i):
                invt = invt_v[...]
                w = wbuf[pl.ds(i * LANES, LANES)]
                us = []
                s = None
                m = None
                for k in range(NCOMP):
                    d = w - mu_v[k]
                    u = coef_v[k] * jnp.exp(d * d * nis_v[k])
                    us.append(u)
                    if k == 0:
                        s = u
                        m = u
                    else:
                        s = s + u
                        m = jnp.maximum(m, u)
                c = invt / (s + EPS)
                den = None
                num = None
                for k in range(NCOMP):
                    e = jnp.exp((us[k] - m) * c)
                    if k == 0:
                        den = e
                    else:
                        den = den + e
                        num = e * mu_v[k] if k == 1 else num + e * mu_v[k]
                obuf[pl.ds(i * LANES, LANES)] = num / den

            pltpu.sync_copy(obuf, out_hbm.at[pl.ds(off, CHUNK)])

    return body(w_flat, mu_b, nis_b, coef_b, invt_b)


def kernel(weights, mu, pi_k, pi_zero, sigma, sigma_zero, temperature):
    w = weights.reshape(-1)
    pi_tmp = jnp.abs(jnp.concatenate([pi_zero, pi_k]))
    pi_norm = pi_tmp / jnp.sum(pi_tmp)
    mu_all = jnp.concatenate([jnp.zeros((1,), weights.dtype), mu])
    sigma_all = jnp.concatenate([sigma_zero, sigma])
    two_sig2 = 2.0 * sigma_all ** 2
    coef = pi_norm / jnp.sqrt(math.pi * two_sig2)
    nis = -1.0 / two_sig2

    mu_b = jnp.broadcast_to(mu_all[:, None], (NCOMP, LANES))
    nis_b = jnp.broadcast_to(nis[:, None], (NCOMP, LANES))
    coef_b = jnp.broadcast_to(coef[:, None], (NCOMP, LANES))
    invt_b = jnp.broadcast_to(1.0 / temperature, (LANES,))

    out = _sc_gmm(w, mu_b, nis_b, coef_b, invt_b)
    return out.reshape(weights.shape)


# speedup vs baseline: 2.0494x; 1.2394x over previous
import functools
import math

import jax
import jax.numpy as jnp
from jax import lax
from jax.experimental import pallas as pl
from jax.experimental.pallas import tpu as pltpu
from jax.experimental.pallas import tpu_sc as plsc

EPS = 1e-8
NCOMP = 16
LANES = 16
NC, NS = 2, 16
NW = NC * NS
N = 1024 * 2048
PER_W = N // NW
CHUNK = 8192
NCHUNK = PER_W // CHUNK
VPC = CHUNK // LANES


def _sc_gmm(w_flat, mu_b, nis_b, coef_b, invt_b):
    mesh = plsc.VectorSubcoreMesh(core_axis_name="c", subcore_axis_name="s")

    @functools.partial(
        pl.kernel,
        mesh=mesh,
        out_type=jax.ShapeDtypeStruct((N,), jnp.float32),
        scratch_types=[
            pltpu.VMEM((NCOMP, LANES), jnp.float32),
            pltpu.VMEM((NCOMP, LANES), jnp.float32),
            pltpu.VMEM((NCOMP, LANES), jnp.float32),
            pltpu.VMEM((LANES,), jnp.float32),
            pltpu.VMEM((CHUNK,), jnp.float32),
            pltpu.VMEM((CHUNK,), jnp.float32),
        ],
    )
    def body(w_hbm, mu_hbm, nis_hbm, coef_hbm, invt_hbm, out_hbm,
             mu_v, nis_v, coef_v, invt_v, wbuf, obuf):
        wid = lax.axis_index("s") * NC + lax.axis_index("c")
        base = wid * PER_W
        pltpu.sync_copy(mu_hbm, mu_v)
        pltpu.sync_copy(nis_hbm, nis_v)
        pltpu.sync_copy(coef_hbm, coef_v)
        pltpu.sync_copy(invt_hbm, invt_v)

        @pl.loop(0, NCHUNK)
        def chunk_body(j):
            off = base + j * CHUNK
            pltpu.sync_copy(w_hbm.at[pl.ds(off, CHUNK)], wbuf)

            @plsc.parallel_loop(0, CHUNK, step=LANES, unroll=2)
            def vec_body(i):
                invt = invt_v[...]
                w = wbuf[pl.ds(i, LANES)]
                us = []
                s = None
                m = None
                for k in range(NCOMP):
                    d = w - mu_v[k]
                    u = jnp.exp(d * d * nis_v[k] + coef_v[k])
                    us.append(u)
                    if k == 0:
                        s = u
                        m = u
                    else:
                        s = s + u
                        m = jnp.maximum(m, u)
                c = invt / (s + EPS)
                mc = m * c
                den = None
                num = None
                for k in range(NCOMP):
                    e = jnp.exp(us[k] * c - mc)
                    if k == 0:
                        den = e
                    else:
                        den = den + e
                        num = e * mu_v[k] if k == 1 else num + e * mu_v[k]
                obuf[pl.ds(i, LANES)] = num / den

            pltpu.sync_copy(obuf, out_hbm.at[pl.ds(off, CHUNK)])

    return body(w_flat, mu_b, nis_b, coef_b, invt_b)


def kernel(weights, mu, pi_k, pi_zero, sigma, sigma_zero, temperature):
    w = weights.reshape(-1)
    pi_tmp = jnp.abs(jnp.concatenate([pi_zero, pi_k]))
    pi_norm = pi_tmp / jnp.sum(pi_tmp)
    mu_all = jnp.concatenate([jnp.zeros((1,), weights.dtype), mu])
    sigma_all = jnp.concatenate([sigma_zero, sigma])
    two_sig2 = 2.0 * sigma_all ** 2
    coef = pi_norm / jnp.sqrt(math.pi * two_sig2)
    nis = -1.0 / two_sig2
    lc = jnp.log(coef)

    mu_b = jnp.broadcast_to(mu_all[:, None], (NCOMP, LANES))
    nis_b = jnp.broadcast_to(nis[:, None], (NCOMP, LANES))
    coef_b = jnp.broadcast_to(lc[:, None], (NCOMP, LANES))
    invt_b = jnp.broadcast_to(1.0 / temperature, (LANES,))

    out = _sc_gmm(w, mu_b, nis_b, coef_b, invt_b)
    return out.reshape(weights.shape)
